# Initial kernel scaffold; baseline (speedup 1.0000x reference)
import jax
import jax.numpy as jnp
from jax.experimental import pallas as pl


def kernel(x, edge_index, W1, b1, W2, b2):
    raise NotImplementedError("write your pallas kernel here")


# speedup vs baseline: 3.7243x; 3.7243x over previous
import functools

import jax
import jax.numpy as jnp
from jax import lax
from jax.experimental import pallas as pl
from jax.experimental.pallas import tpu as pltpu
from jax.experimental.pallas import tpu_sc as plsc

_N = 10000
_D = 128
_E = 320000
_H = 256
_O = 16

_NC = 2
_NS = 16
_NW = _NC * _NS
_C = 128
_NCHUNK = 80
_EW = _C * _NCHUNK
_EPAD = _EW * _NW
_NPAD = 10240
_ROWS_PER_SUB = _NPAD // _NS


def _layer_body(h_hbm, src_hbm, dst_hbm, out_hbm, src_v, dst_v, rows_v, acc_sh, sem):
    c = lax.axis_index("c")
    s = lax.axis_index("s")
    wid = c * _NS + s
    pltpu.sync_copy(src_hbm.at[wid], src_v)
    pltpu.sync_copy(dst_hbm.at[wid], dst_v)
    pltpu.sync_copy(
        h_hbm.at[pl.ds(s * _ROWS_PER_SUB, _ROWS_PER_SUB)],
        acc_sh.at[pl.ds(s * _ROWS_PER_SUB, _ROWS_PER_SUB)],
    )
    plsc.subcore_barrier()

    def body(j, carry):
        pltpu.async_copy(h_hbm.at[src_v.at[j]], rows_v, sem).wait()
        pltpu.sync_copy(rows_v, acc_sh.at[dst_v.at[j]], add=True)
        return carry

    lax.fori_loop(0, _NCHUNK, body, 0)
    plsc.subcore_barrier()
    pltpu.sync_copy(
        acc_sh.at[pl.ds(s * _ROWS_PER_SUB, _ROWS_PER_SUB)],
        out_hbm.at[c, pl.ds(s * _ROWS_PER_SUB, _ROWS_PER_SUB)],
    )


_layer = pl.kernel(
    _layer_body,
    mesh=plsc.VectorSubcoreMesh(core_axis_name="c", subcore_axis_name="s"),
    out_type=jax.ShapeDtypeStruct((_NC, _NPAD, _D), jnp.float32),
    scratch_types=[
        pltpu.VMEM((_NCHUNK, _C), jnp.int32),
        pltpu.VMEM((_NCHUNK, _C), jnp.int32),
        pltpu.VMEM((_C, _D), jnp.float32),
        pltpu.VMEM_SHARED((_NPAD, _D), jnp.float32),
        pltpu.SemaphoreType.DMA,
    ],
)


_B = 1024


def _combine_body(p_ref, x_ref, o_ref):
    o_ref[...] = p_ref[0] + p_ref[1] - x_ref[...]


def _combine(p, x):
    return pl.pallas_call(
        _combine_body,
        grid=(_NPAD // _B,),
        in_specs=[
            pl.BlockSpec((_NC, _B, _D), lambda i: (0, i, 0)),
            pl.BlockSpec((_B, _D), lambda i: (i, 0)),
        ],
        out_specs=pl.BlockSpec((_B, _D), lambda i: (i, 0)),
        out_shape=jax.ShapeDtypeStruct((_NPAD, _D), jnp.float32),
    )(p, x)


def _mlp_body(q_ref, h1_ref, w1_ref, b1_ref, w2_ref, b2_ref, o_ref):
    h2 = q_ref[0] + q_ref[1] - h1_ref[...]
    t = jnp.dot(h2, w1_ref[...], preferred_element_type=jnp.float32) + b1_ref[...]
    t = jnp.maximum(t, 0.0)
    o_ref[...] = jnp.dot(t, w2_ref[...], preferred_element_type=jnp.float32) + b2_ref[...]


def _mlp(q, h1, w1, b1, w2, b2):
    return pl.pallas_call(
        _mlp_body,
        grid=(_NPAD // _B,),
        in_specs=[
            pl.BlockSpec((_NC, _B, _D), lambda i: (0, i, 0)),
            pl.BlockSpec((_B, _D), lambda i: (i, 0)),
            pl.BlockSpec((_D, _H), lambda i: (0, 0)),
            pl.BlockSpec((1, _H), lambda i: (0, 0)),
            pl.BlockSpec((_H, _O), lambda i: (0, 0)),
            pl.BlockSpec((1, _O), lambda i: (0, 0)),
        ],
        out_specs=pl.BlockSpec((_B, _O), lambda i: (i, 0)),
        out_shape=jax.ShapeDtypeStruct((_NPAD, _O), jnp.float32),
    )(q, h1, w1, b1, w2, b2)


def kernel(x, edge_index, W1, b1, W2, b2):
    pad = _EPAD - _E
    src = jnp.concatenate([edge_index[0], jnp.zeros((pad,), jnp.int32)])
    dst = jnp.concatenate([edge_index[1], jnp.full((pad,), _N, jnp.int32)])
    src = src.reshape(_NW, _NCHUNK, _C)
    dst = dst.reshape(_NW, _NCHUNK, _C)
    xp = jnp.concatenate([x, jnp.zeros((_NPAD - _N, _D), jnp.float32)])

    p = _layer(xp, src, dst)
    h1 = _combine(p, xp)
    q = _layer(h1, src, dst)
    out = _mlp(q, h1, W1, b1.reshape(1, _H), W2, b2.reshape(1, _O))
    return out[:_N]


# speedup vs baseline: 4.3147x; 1.1585x over previous
import functools

import jax
import jax.numpy as jnp
from jax import lax
from jax.experimental import pallas as pl
from jax.experimental.pallas import tpu as pltpu
from jax.experimental.pallas import tpu_sc as plsc

_N = 10000
_D = 128
_E = 320000
_H = 256
_O = 16

_NC = 2
_NS = 16
_NW = _NC * _NS
_C = 80
_NCHUNK = 128
_NSTAGE = 2
_SCHUNK = _NCHUNK // _NSTAGE
_EW = _C * _NCHUNK
_EPAD = _EW * _NW
_NPAD = 10240
_ROWS_PER_SUB = _NPAD // _NS


def _layer_body(h_hbm, src_hbm, dst_hbm, out_hbm, src_v, dst_v, rows0, rows1,
                acc_sh, sem0, sem1):
    c = lax.axis_index("c")
    s = lax.axis_index("s")
    wid = c * _NS + s
    pltpu.sync_copy(
        h_hbm.at[pl.ds(s * _ROWS_PER_SUB, _ROWS_PER_SUB)],
        acc_sh.at[pl.ds(s * _ROWS_PER_SUB, _ROWS_PER_SUB)],
    )
    plsc.subcore_barrier()

    for stage in range(_NSTAGE):
        pltpu.sync_copy(src_hbm.at[wid, pl.ds(stage * _SCHUNK, _SCHUNK)], src_v)
        pltpu.sync_copy(dst_hbm.at[wid, pl.ds(stage * _SCHUNK, _SCHUNK)], dst_v)
        pltpu.async_copy(h_hbm.at[src_v.at[0]], rows0, sem0)

        def body(jj, carry):
            j0 = 2 * jj
            j1 = 2 * jj + 1
            pltpu.async_copy(h_hbm.at[src_v.at[j1]], rows1, sem1)
            pltpu.make_async_copy(h_hbm.at[src_v.at[j0]], rows0, sem0).wait()
            pltpu.sync_copy(rows0, acc_sh.at[dst_v.at[j0]], add=True)

            @pl.when(jj < _SCHUNK // 2 - 1)
            def _():
                pltpu.async_copy(h_hbm.at[src_v.at[j0 + 2]], rows0, sem0)

            pltpu.make_async_copy(h_hbm.at[src_v.at[j1]], rows1, sem1).wait()
            pltpu.sync_copy(rows1, acc_sh.at[dst_v.at[j1]], add=True)
            return carry

        lax.fori_loop(0, _SCHUNK // 2, body, 0)
    plsc.subcore_barrier()
    pltpu.sync_copy(
        acc_sh.at[pl.ds(s * _ROWS_PER_SUB, _ROWS_PER_SUB)],
        out_hbm.at[c, pl.ds(s * _ROWS_PER_SUB, _ROWS_PER_SUB)],
    )


_layer = pl.kernel(
    _layer_body,
    mesh=plsc.VectorSubcoreMesh(core_axis_name="c", subcore_axis_name="s"),
    out_type=jax.ShapeDtypeStruct((_NC, _NPAD, _D), jnp.float32),
    scratch_types=[
        pltpu.VMEM((_SCHUNK, _C), jnp.int32),
        pltpu.VMEM((_SCHUNK, _C), jnp.int32),
        pltpu.VMEM((_C, _D), jnp.float32),
        pltpu.VMEM((_C, _D), jnp.float32),
        pltpu.VMEM_SHARED((_NPAD, _D), jnp.float32),
        pltpu.SemaphoreType.DMA,
        pltpu.SemaphoreType.DMA,
    ],
)


_B = 1024


def _combine_body(p_ref, x_ref, o_ref):
    o_ref[...] = p_ref[0] + p_ref[1] - x_ref[...]


def _combine(p, x):
    return pl.pallas_call(
        _combine_body,
        grid=(_NPAD // _B,),
        in_specs=[
            pl.BlockSpec((_NC, _B, _D), lambda i: (0, i, 0)),
            pl.BlockSpec((_B, _D), lambda i: (i, 0)),
        ],
        out_specs=pl.BlockSpec((_B, _D), lambda i: (i, 0)),
        out_shape=jax.ShapeDtypeStruct((_NPAD, _D), jnp.float32),
    )(p, x)


def _mlp_body(q_ref, h1_ref, w1_ref, b1_ref, w2_ref, b2_ref, o_ref):
    h2 = q_ref[0] + q_ref[1] - h1_ref[...]
    t = jnp.dot(h2, w1_ref[...], preferred_element_type=jnp.float32) + b1_ref[...]
    t = jnp.maximum(t, 0.0)
    o_ref[...] = jnp.dot(t, w2_ref[...], preferred_element_type=jnp.float32) + b2_ref[...]


def _mlp(q, h1, w1, b1, w2, b2):
    return pl.pallas_call(
        _mlp_body,
        grid=(_NPAD // _B,),
        in_specs=[
            pl.BlockSpec((_NC, _B, _D), lambda i: (0, i, 0)),
            pl.BlockSpec((_B, _D), lambda i: (i, 0)),
            pl.BlockSpec((_D, _H), lambda i: (0, 0)),
            pl.BlockSpec((1, _H), lambda i: (0, 0)),
            pl.BlockSpec((_H, _O), lambda i: (0, 0)),
            pl.BlockSpec((1, _O), lambda i: (0, 0)),
        ],
        out_specs=pl.BlockSpec((_B, _O), lambda i: (i, 0)),
        out_shape=jax.ShapeDtypeStruct((_NPAD, _O), jnp.float32),
    )(q, h1, w1, b1, w2, b2)


def kernel(x, edge_index, W1, b1, W2, b2):
    pad = _EPAD - _E
    src = jnp.concatenate([edge_index[0], jnp.zeros((pad,), jnp.int32)])
    dst = jnp.concatenate([edge_index[1], jnp.full((pad,), _N, jnp.int32)])
    src = src.reshape(_NW, _NCHUNK, _C)
    dst = dst.reshape(_NW, _NCHUNK, _C)
    xp = jnp.concatenate([x, jnp.zeros((_NPAD - _N, _D), jnp.float32)])

    p = _layer(xp, src, dst)
    h1 = _combine(p, xp)
    q = _layer(h1, src, dst)
    out = _mlp(q, h1, W1, b1.reshape(1, _H), W2, b2.reshape(1, _O))
    return out[:_N]
